# baseline (speedup 1.0000x reference)
import functools

import jax
import jax.numpy as jnp
from jax import lax
from jax.experimental import pallas as pl
from jax.experimental.pallas import tpu as pltpu
from jax.experimental.pallas import tpu_sc as plsc

N = 10000
NP = 10240
D = 128
E = 320000
NC, NS = 2, 16
NW = NC * NS
CHUNK = 64
CPT = 160
E_TILE = CPT * CHUNK
E_PAD = E_TILE * NW
NBUF = 4
WIN = 16
ROWS_T = NP // NS
RB = 1024
GRID = NP // RB

_mesh = plsc.VectorSubcoreMesh(
    core_axis_name="c", subcore_axis_name="s", num_cores=NC, num_subcores=NS)


def _splat_lane(vec, lane):
    idx = jnp.full((16, 1), lane, jnp.int32)
    return lax.gather(
        vec, idx,
        dimension_numbers=lax.GatherDimensionNumbers(
            offset_dims=(), collapsed_slice_dims=(0,), start_index_map=(0,)),
        slice_sizes=(1,),
        mode=lax.GatherScatterMode.PROMISE_IN_BOUNDS)


def _zero_rows(buf, nrows):
    @pl.loop(0, nrows)
    def _(r):
        for cc in range(D // 16):
            buf[r, pl.ds(cc * 16, 16)] = jnp.zeros((16,), jnp.float32)


@functools.partial(
    pl.kernel,
    out_type=jax.ShapeDtypeStruct((NP, D), jnp.float32),
    mesh=_mesh,
    scratch_types=[
        pltpu.VMEM((CPT, CHUNK), jnp.int32),
        pltpu.VMEM((CPT, CHUNK), jnp.float32),
        pltpu.VMEM((ROWS_T,), jnp.float32),
        pltpu.VMEM((CHUNK, D), jnp.float32),
        pltpu.VMEM_SHARED((NP,), jnp.float32),
    ],
)
def _deg_dinv(dstT, wT, deg_out, dst_v, w_v, deg_v, bcast_v, deg_acc):
    c = lax.axis_index("c")
    s = lax.axis_index("s")

    @pl.when(c == 0)
    def _():
        @pl.loop(0, ROWS_T // 16)
        def _(k):
            deg_v[pl.ds(k * 16, 16)] = jnp.zeros((16,), jnp.float32)
        pltpu.sync_copy(deg_v, deg_acc.at[pl.ds(s * ROWS_T, ROWS_T)])

    plsc.subcore_barrier()

    @pl.when(c == 0)
    def _():
        for half in range(2):
            wid = s * 2 + half
            pltpu.sync_copy(dstT.at[wid], dst_v)
            pltpu.sync_copy(wT.at[wid], w_v)

            @pl.loop(0, CPT)
            def _(j):
                pltpu.sync_copy(w_v.at[j], deg_acc.at[dst_v.at[j]], add=True)

    plsc.subcore_barrier()

    @pl.when(c == 0)
    def _():
        pltpu.sync_copy(deg_acc.at[pl.ds(s * ROWS_T, ROWS_T)], deg_v)
        @pl.loop(0, ROWS_T // CHUNK)
        def _(b):
            @pl.loop(0, CHUNK // 16)
            def _(rb):
                dvec = deg_v[pl.ds(b * CHUNK + rb * 16, 16)]
                for lane in range(16):
                    row = jnp.full((16,), dvec[lane], jnp.float32)
                    for cc in range(D // 16):
                        bcast_v[rb * 16 + lane, pl.ds(cc * 16, 16)] = row
            pltpu.sync_copy(
                bcast_v, deg_out.at[pl.ds(s * ROWS_T + b * CHUNK, CHUNK)])


@functools.partial(
    pl.kernel,
    out_type=jax.ShapeDtypeStruct((NC, NP, D), jnp.float32),
    mesh=_mesh,
    scratch_types=[
        pltpu.VMEM((WIN, CHUNK), jnp.int32),
        pltpu.VMEM((WIN, CHUNK), jnp.int32),
        pltpu.VMEM((WIN, CHUNK), jnp.float32),
        [pltpu.VMEM((CHUNK, D), jnp.float32) for _ in range(NBUF)],
        pltpu.VMEM_SHARED((NP, D), jnp.float32),
        [pltpu.SemaphoreType.DMA for _ in range(NBUF)],
    ],
)
def _agg(g, srcT, dstT, wT, part, src_v, dst_v, w_v, bufs, acc, sems):
    c = lax.axis_index("c")
    s = lax.axis_index("s")
    wid = s * NC + c

    _zero_rows(bufs[0], CHUNK)

    @pl.loop(0, ROWS_T // CHUNK)
    def _(k):
        pltpu.sync_copy(bufs[0], acc.at[pl.ds(s * ROWS_T + k * CHUNK, CHUNK)])

    plsc.subcore_barrier()

    @pl.loop(0, CPT // WIN)
    def _(w):
        wsl = pl.ds(w * WIN, WIN)
        pltpu.sync_copy(srcT.at[wid, wsl], src_v)
        pltpu.sync_copy(dstT.at[wid, wsl], dst_v)
        pltpu.sync_copy(wT.at[wid, wsl], w_v)

        for b in range(NBUF):
            pltpu.async_copy(g.at[src_v.at[b]], bufs[b], sems[b])

        @pl.loop(0, WIN // NBUF)
        def _(q):
            for b in range(NBUF):
                jl = q * NBUF + b
                buf = bufs[b]
                pltpu.make_async_copy(g.at[src_v.at[jl]], buf, sems[b]).wait()

                @pl.loop(0, CHUNK // 16)
                def _(eb):
                    wvec = w_v[jl, pl.ds(eb * 16, 16)]
                    for lane in range(16):
                        we = wvec[lane]
                        for cc in range(D // 16):
                            sl = pl.ds(cc * 16, 16)
                            buf[eb * 16 + lane, sl] = buf[eb * 16 + lane, sl] * we

                pltpu.sync_copy(buf, acc.at[dst_v.at[jl]], add=True)

                @pl.when(jl + NBUF < WIN)
                def _():
                    pltpu.async_copy(g.at[src_v.at[jl + NBUF]], buf, sems[b])

    plsc.subcore_barrier()

    @pl.loop(0, ROWS_T // CHUNK)
    def _(k):
        pltpu.sync_copy(acc.at[pl.ds(s * ROWS_T + k * CHUNK, CHUNK)], bufs[0])
        pltpu.sync_copy(
            bufs[0], part.at[c, pl.ds(s * ROWS_T + k * CHUNK, CHUNK)])


def _mm_scale_body(x_ref, w_ref, deg_ref, o_ref):
    dinv = lax.rsqrt(deg_ref[...] + 1.0)
    o_ref[...] = dinv * jnp.dot(
        x_ref[...], w_ref[...], preferred_element_type=jnp.float32)


_mm_scale = pl.pallas_call(
    _mm_scale_body,
    grid=(GRID,),
    in_specs=[
        pl.BlockSpec((RB, D), lambda i: (i, 0)),
        pl.BlockSpec((D, D), lambda i: (0, 0)),
        pl.BlockSpec((RB, D), lambda i: (i, 0)),
    ],
    out_specs=pl.BlockSpec((RB, D), lambda i: (i, 0)),
    out_shape=jax.ShapeDtypeStruct((NP, D), jnp.float32),
)


def _combine_mm_body(p_ref, g_ref, deg_ref, b_ref, w_ref, h_ref, g2_ref):
    dinv = lax.rsqrt(deg_ref[...] + 1.0)
    h = jnp.maximum(
        dinv * (p_ref[0] + p_ref[1] + g_ref[...]) + b_ref[...], 0.0)
    h_ref[...] = h
    g2_ref[...] = dinv * jnp.dot(
        h, w_ref[...], preferred_element_type=jnp.float32)


_combine_mm = pl.pallas_call(
    _combine_mm_body,
    grid=(GRID,),
    in_specs=[
        pl.BlockSpec((NC, RB, D), lambda i: (0, i, 0)),
        pl.BlockSpec((RB, D), lambda i: (i, 0)),
        pl.BlockSpec((RB, D), lambda i: (i, 0)),
        pl.BlockSpec((1, D), lambda i: (0, 0)),
        pl.BlockSpec((D, D), lambda i: (0, 0)),
    ],
    out_specs=[
        pl.BlockSpec((RB, D), lambda i: (i, 0)),
        pl.BlockSpec((RB, D), lambda i: (i, 0)),
    ],
    out_shape=[
        jax.ShapeDtypeStruct((NP, D), jnp.float32),
        jax.ShapeDtypeStruct((NP, D), jnp.float32),
    ],
)


def _final_body(p_ref, g_ref, deg_ref, b_ref, h1_ref, o_ref):
    dinv = lax.rsqrt(deg_ref[...] + 1.0)
    h2 = jnp.maximum(
        dinv * (p_ref[0] + p_ref[1] + g_ref[...]) + b_ref[...], 0.0)
    o_ref[...] = (h1_ref[...] + h2) * 0.5


_final = pl.pallas_call(
    _final_body,
    grid=(GRID,),
    in_specs=[
        pl.BlockSpec((NC, RB, D), lambda i: (0, i, 0)),
        pl.BlockSpec((RB, D), lambda i: (i, 0)),
        pl.BlockSpec((RB, D), lambda i: (i, 0)),
        pl.BlockSpec((1, D), lambda i: (0, 0)),
        pl.BlockSpec((RB, D), lambda i: (i, 0)),
    ],
    out_specs=pl.BlockSpec((RB, D), lambda i: (i, 0)),
    out_shape=jax.ShapeDtypeStruct((NP, D), jnp.float32),
)


def kernel(x, edge_index, edge_weight, W1, b1, W2, b2):
    pad = E_PAD - E
    pad_ids = jnp.arange(pad, dtype=jnp.int32) % N
    srcT = jnp.concatenate([edge_index[0], pad_ids]).reshape(NW, CPT, CHUNK)
    dstT = jnp.concatenate([edge_index[1], pad_ids]).reshape(NW, CPT, CHUNK)
    wT = jnp.pad(edge_weight, (0, pad)).reshape(NW, CPT, CHUNK)
    x_pad = jnp.pad(x, ((0, NP - N), (0, 0)))
    b1r = b1.reshape(1, D)
    b2r = b2.reshape(1, D)

    deg_b = _deg_dinv(dstT, wT)
    g1 = _mm_scale(x_pad, W1, deg_b)
    p1 = _agg(g1, srcT, dstT, wT)
    h1, g2 = _combine_mm(p1, g1, deg_b, b1r, W2)
    p2 = _agg(g2, srcT, dstT, wT)
    out = _final(p2, g2, deg_b, b2r, h1)
    return out[:N]


# speedup vs baseline: 1.2170x; 1.2170x over previous
import functools

import jax
import jax.numpy as jnp
from jax import lax
from jax.experimental import pallas as pl
from jax.experimental.pallas import tpu as pltpu
from jax.experimental.pallas import tpu_sc as plsc

N = 10000
NP = 10240
D = 128
E = 320000
NC, NS = 2, 16
NW = NC * NS
CHUNK = 128
CPT = 80
E_TILE = CPT * CHUNK
E_PAD = E_TILE * NW
NBUF = 2
WIN = 40
ROWS_T = NP // NS
RB = 1024
GRID = NP // RB

_mesh = plsc.VectorSubcoreMesh(
    core_axis_name="c", subcore_axis_name="s", num_cores=NC, num_subcores=NS)


def _splat_lane(vec, lane):
    idx = jnp.full((16, 1), lane, jnp.int32)
    return lax.gather(
        vec, idx,
        dimension_numbers=lax.GatherDimensionNumbers(
            offset_dims=(), collapsed_slice_dims=(0,), start_index_map=(0,)),
        slice_sizes=(1,),
        mode=lax.GatherScatterMode.PROMISE_IN_BOUNDS)


def _zero_rows(buf, nrows):
    @pl.loop(0, nrows)
    def _(r):
        for cc in range(D // 16):
            buf[r, pl.ds(cc * 16, 16)] = jnp.zeros((16,), jnp.float32)


@functools.partial(
    pl.kernel,
    out_type=jax.ShapeDtypeStruct((NC, NP, D), jnp.float32),
    mesh=_mesh,
    scratch_types=[
        pltpu.VMEM((CPT, CHUNK), jnp.int32),
        pltpu.VMEM((CPT, CHUNK), jnp.float32),
        pltpu.VMEM((ROWS_T,), jnp.float32),
        pltpu.VMEM((CHUNK, D), jnp.float32),
        pltpu.VMEM_SHARED((NP,), jnp.float32),
    ],
)
def _deg_dinv(dstT, wT, deg_out, dst_v, w_v, deg_v, bcast_v, deg_acc):
    c = lax.axis_index("c")
    s = lax.axis_index("s")
    wid = s * NC + c

    @pl.loop(0, ROWS_T // 16)
    def _(k):
        deg_v[pl.ds(k * 16, 16)] = jnp.zeros((16,), jnp.float32)
    pltpu.sync_copy(deg_v, deg_acc.at[pl.ds(s * ROWS_T, ROWS_T)])

    plsc.subcore_barrier()

    pltpu.sync_copy(dstT.at[wid], dst_v)
    pltpu.sync_copy(wT.at[wid], w_v)

    @pl.loop(0, CPT)
    def _(j):
        pltpu.sync_copy(w_v.at[j], deg_acc.at[dst_v.at[j]], add=True)

    plsc.subcore_barrier()

    pltpu.sync_copy(deg_acc.at[pl.ds(s * ROWS_T, ROWS_T)], deg_v)
    @pl.loop(0, ROWS_T // CHUNK)
    def _(b):
        @pl.loop(0, CHUNK // 16)
        def _(rb):
            dvec = deg_v[pl.ds(b * CHUNK + rb * 16, 16)]
            for lane in range(16):
                row = jnp.full((16,), dvec[lane], jnp.float32)
                for cc in range(D // 16):
                    bcast_v[rb * 16 + lane, pl.ds(cc * 16, 16)] = row
        pltpu.sync_copy(
            bcast_v,
            deg_out.at[c, pl.ds(s * ROWS_T + b * CHUNK, CHUNK)])


@functools.partial(
    pl.kernel,
    out_type=jax.ShapeDtypeStruct((NC, NP, D), jnp.float32),
    mesh=_mesh,
    scratch_types=[
        pltpu.VMEM((WIN, CHUNK), jnp.int32),
        pltpu.VMEM((WIN, CHUNK), jnp.int32),
        pltpu.VMEM((WIN, CHUNK), jnp.float32),
        [pltpu.VMEM((CHUNK, D), jnp.float32) for _ in range(NBUF)],
        pltpu.VMEM_SHARED((NP, D), jnp.float32),
        [pltpu.SemaphoreType.DMA for _ in range(NBUF)],
    ],
)
def _agg(g, srcT, dstT, wT, part, src_v, dst_v, w_v, bufs, acc, sems):
    c = lax.axis_index("c")
    s = lax.axis_index("s")
    wid = s * NC + c

    _zero_rows(bufs[0], CHUNK)

    @pl.loop(0, ROWS_T // CHUNK)
    def _(k):
        pltpu.sync_copy(bufs[0], acc.at[pl.ds(s * ROWS_T + k * CHUNK, CHUNK)])

    plsc.subcore_barrier()

    @pl.loop(0, CPT // WIN)
    def _(w):
        wsl = pl.ds(w * WIN, WIN)
        pltpu.sync_copy(srcT.at[wid, wsl], src_v)
        pltpu.sync_copy(dstT.at[wid, wsl], dst_v)
        pltpu.sync_copy(wT.at[wid, wsl], w_v)

        for b in range(NBUF):
            pltpu.async_copy(g.at[src_v.at[b]], bufs[b], sems[b])

        @pl.loop(0, WIN // NBUF)
        def _(q):
            for b in range(NBUF):
                jl = q * NBUF + b
                buf = bufs[b]
                pltpu.make_async_copy(g.at[src_v.at[jl]], buf, sems[b]).wait()

                @pl.loop(0, CHUNK // 16)
                def _(eb):
                    wvec = w_v[jl, pl.ds(eb * 16, 16)]
                    for lane in range(16):
                        we = wvec[lane]
                        for cc in range(D // 16):
                            sl = pl.ds(cc * 16, 16)
                            buf[eb * 16 + lane, sl] = buf[eb * 16 + lane, sl] * we

                pltpu.sync_copy(buf, acc.at[dst_v.at[jl]], add=True)

                @pl.when(jl + NBUF < WIN)
                def _():
                    pltpu.async_copy(g.at[src_v.at[jl + NBUF]], buf, sems[b])

    plsc.subcore_barrier()

    @pl.loop(0, ROWS_T // CHUNK)
    def _(k):
        pltpu.sync_copy(acc.at[pl.ds(s * ROWS_T + k * CHUNK, CHUNK)], bufs[0])
        pltpu.sync_copy(
            bufs[0], part.at[c, pl.ds(s * ROWS_T + k * CHUNK, CHUNK)])


def _mm_scale_body(x_ref, w_ref, deg_ref, o_ref):
    dinv = lax.rsqrt(deg_ref[0] + deg_ref[1] + 1.0)
    o_ref[...] = dinv * jnp.dot(
        x_ref[...], w_ref[...], preferred_element_type=jnp.float32)


_mm_scale = pl.pallas_call(
    _mm_scale_body,
    grid=(GRID,),
    in_specs=[
        pl.BlockSpec((RB, D), lambda i: (i, 0)),
        pl.BlockSpec((D, D), lambda i: (0, 0)),
        pl.BlockSpec((NC, RB, D), lambda i: (0, i, 0)),
    ],
    out_specs=pl.BlockSpec((RB, D), lambda i: (i, 0)),
    out_shape=jax.ShapeDtypeStruct((NP, D), jnp.float32),
)


def _combine_mm_body(p_ref, g_ref, deg_ref, b_ref, w_ref, h_ref, g2_ref):
    dinv = lax.rsqrt(deg_ref[0] + deg_ref[1] + 1.0)
    h = jnp.maximum(
        dinv * (p_ref[0] + p_ref[1] + g_ref[...]) + b_ref[...], 0.0)
    h_ref[...] = h
    g2_ref[...] = dinv * jnp.dot(
        h, w_ref[...], preferred_element_type=jnp.float32)


_combine_mm = pl.pallas_call(
    _combine_mm_body,
    grid=(GRID,),
    in_specs=[
        pl.BlockSpec((NC, RB, D), lambda i: (0, i, 0)),
        pl.BlockSpec((RB, D), lambda i: (i, 0)),
        pl.BlockSpec((NC, RB, D), lambda i: (0, i, 0)),
        pl.BlockSpec((1, D), lambda i: (0, 0)),
        pl.BlockSpec((D, D), lambda i: (0, 0)),
    ],
    out_specs=[
        pl.BlockSpec((RB, D), lambda i: (i, 0)),
        pl.BlockSpec((RB, D), lambda i: (i, 0)),
    ],
    out_shape=[
        jax.ShapeDtypeStruct((NP, D), jnp.float32),
        jax.ShapeDtypeStruct((NP, D), jnp.float32),
    ],
)


def _final_body(p_ref, g_ref, deg_ref, b_ref, h1_ref, o_ref):
    dinv = lax.rsqrt(deg_ref[0] + deg_ref[1] + 1.0)
    h2 = jnp.maximum(
        dinv * (p_ref[0] + p_ref[1] + g_ref[...]) + b_ref[...], 0.0)
    o_ref[...] = (h1_ref[...] + h2) * 0.5


_final = pl.pallas_call(
    _final_body,
    grid=(GRID,),
    in_specs=[
        pl.BlockSpec((NC, RB, D), lambda i: (0, i, 0)),
        pl.BlockSpec((RB, D), lambda i: (i, 0)),
        pl.BlockSpec((NC, RB, D), lambda i: (0, i, 0)),
        pl.BlockSpec((1, D), lambda i: (0, 0)),
        pl.BlockSpec((RB, D), lambda i: (i, 0)),
    ],
    out_specs=pl.BlockSpec((RB, D), lambda i: (i, 0)),
    out_shape=jax.ShapeDtypeStruct((NP, D), jnp.float32),
)


def kernel(x, edge_index, edge_weight, W1, b1, W2, b2):
    pad = E_PAD - E
    pad_ids = jnp.arange(pad, dtype=jnp.int32) % N
    srcT = jnp.concatenate([edge_index[0], pad_ids]).reshape(NW, CPT, CHUNK)
    dstT = jnp.concatenate([edge_index[1], pad_ids]).reshape(NW, CPT, CHUNK)
    wT = jnp.pad(edge_weight, (0, pad)).reshape(NW, CPT, CHUNK)
    x_pad = jnp.pad(x, ((0, NP - N), (0, 0)))
    b1r = b1.reshape(1, D)
    b2r = b2.reshape(1, D)

    deg_b = _deg_dinv(dstT, wT)
    g1 = _mm_scale(x_pad, W1, deg_b)
    p1 = _agg(g1, srcT, dstT, wT)
    h1, g2 = _combine_mm(p1, g1, deg_b, b1r, W2)
    p2 = _agg(g2, srcT, dstT, wT)
    out = _final(p2, g2, deg_b, b2r, h1)
    return out[:N]


# speedup vs baseline: 1.2466x; 1.0243x over previous
import functools

import jax
import jax.numpy as jnp
from jax import lax
from jax.experimental import pallas as pl
from jax.experimental.pallas import tpu as pltpu
from jax.experimental.pallas import tpu_sc as plsc

N = 10000
NP = 10240
D = 128
E = 320000
NC, NS = 2, 16
NW = NC * NS
CHUNK = 64
CPT = 160
E_TILE = CPT * CHUNK
E_PAD = E_TILE * NW
NBUF = 4
WIN = 40
ROWS_T = NP // NS
RB = 1024
GRID = NP // RB

_mesh = plsc.VectorSubcoreMesh(
    core_axis_name="c", subcore_axis_name="s", num_cores=NC, num_subcores=NS)


def _splat_lane(vec, lane):
    idx = jnp.full((16, 1), lane, jnp.int32)
    return lax.gather(
        vec, idx,
        dimension_numbers=lax.GatherDimensionNumbers(
            offset_dims=(), collapsed_slice_dims=(0,), start_index_map=(0,)),
        slice_sizes=(1,),
        mode=lax.GatherScatterMode.PROMISE_IN_BOUNDS)


def _zero_rows(buf, nrows):
    @pl.loop(0, nrows)
    def _(r):
        for cc in range(D // 16):
            buf[r, pl.ds(cc * 16, 16)] = jnp.zeros((16,), jnp.float32)


@functools.partial(
    pl.kernel,
    out_type=jax.ShapeDtypeStruct((NC, NP, D), jnp.float32),
    mesh=_mesh,
    scratch_types=[
        pltpu.VMEM((CPT, CHUNK), jnp.int32),
        pltpu.VMEM((CPT, CHUNK), jnp.float32),
        pltpu.VMEM((ROWS_T,), jnp.float32),
        pltpu.VMEM((CHUNK, D), jnp.float32),
        pltpu.VMEM_SHARED((NP,), jnp.float32),
    ],
)
def _deg_dinv(dstT, wT, deg_out, dst_v, w_v, deg_v, bcast_v, deg_acc):
    c = lax.axis_index("c")
    s = lax.axis_index("s")
    wid = s * NC + c

    @pl.loop(0, ROWS_T // 16)
    def _(k):
        deg_v[pl.ds(k * 16, 16)] = jnp.zeros((16,), jnp.float32)
    pltpu.sync_copy(deg_v, deg_acc.at[pl.ds(s * ROWS_T, ROWS_T)])

    plsc.subcore_barrier()

    pltpu.sync_copy(dstT.at[wid], dst_v)
    pltpu.sync_copy(wT.at[wid], w_v)

    @pl.loop(0, CPT)
    def _(j):
        pltpu.sync_copy(w_v.at[j], deg_acc.at[dst_v.at[j]], add=True)

    plsc.subcore_barrier()

    pltpu.sync_copy(deg_acc.at[pl.ds(s * ROWS_T, ROWS_T)], deg_v)
    @pl.loop(0, ROWS_T // CHUNK)
    def _(b):
        @pl.loop(0, CHUNK // 16)
        def _(rb):
            dvec = deg_v[pl.ds(b * CHUNK + rb * 16, 16)]
            for lane in range(16):
                row = jnp.full((16,), dvec[lane], jnp.float32)
                for cc in range(D // 16):
                    bcast_v[rb * 16 + lane, pl.ds(cc * 16, 16)] = row
        pltpu.sync_copy(
            bcast_v,
            deg_out.at[c, pl.ds(s * ROWS_T + b * CHUNK, CHUNK)])


@functools.partial(
    pl.kernel,
    out_type=jax.ShapeDtypeStruct((NC, NP, D), jnp.float32),
    mesh=_mesh,
    scratch_types=[
        pltpu.VMEM((WIN, CHUNK), jnp.int32),
        pltpu.VMEM((WIN, CHUNK), jnp.int32),
        pltpu.VMEM((WIN, CHUNK), jnp.float32),
        [pltpu.VMEM((CHUNK, D), jnp.float32) for _ in range(NBUF)],
        pltpu.VMEM_SHARED((NP, D), jnp.float32),
        [pltpu.SemaphoreType.DMA for _ in range(NBUF)],
        pltpu.SemaphoreType.DMA,
    ],
)
def _agg(g, srcT, dstT, wT, part, src_v, dst_v, w_v, bufs, acc, gsems, ssem):
    c = lax.axis_index("c")
    s = lax.axis_index("s")
    wid = s * NC + c

    _zero_rows(bufs[0], CHUNK)

    @pl.loop(0, ROWS_T // CHUNK)
    def _(k):
        pltpu.sync_copy(bufs[0], acc.at[pl.ds(s * ROWS_T + k * CHUNK, CHUNK)])

    plsc.subcore_barrier()

    @pl.loop(0, CPT // WIN)
    def _(w):
        wsl = pl.ds(w * WIN, WIN)
        pltpu.sync_copy(srcT.at[wid, wsl], src_v)
        pltpu.sync_copy(dstT.at[wid, wsl], dst_v)
        pltpu.sync_copy(wT.at[wid, wsl], w_v)

        for b in range(NBUF - 1):
            pltpu.async_copy(g.at[src_v.at[b]], bufs[b], gsems[b])

        @pl.loop(0, WIN // NBUF)
        def _(q):
            for b in range(NBUF):
                jl = q * NBUF + b
                buf = bufs[b]
                pltpu.make_async_copy(g.at[src_v.at[jl]], buf, gsems[b]).wait()

                @pl.loop(0, CHUNK // 16)
                def _(eb):
                    wvec = w_v[jl, pl.ds(eb * 16, 16)]
                    for lane in range(16):
                        we = wvec[lane]
                        for cc in range(D // 16):
                            sl = pl.ds(cc * 16, 16)
                            buf[eb * 16 + lane, sl] = buf[eb * 16 + lane, sl] * we

                pltpu.async_copy(buf, acc.at[dst_v.at[jl]], ssem, add=True)

                @pl.when(jl + NBUF - 1 < WIN)
                def _():
                    @pl.when(jl >= 1)
                    def _():
                        pltpu.make_async_copy(
                            bufs[0], acc.at[dst_v.at[0]], ssem).wait()
                    pb = (b + NBUF - 1) % NBUF
                    pltpu.async_copy(
                        g.at[src_v.at[jl + NBUF - 1]], bufs[pb], gsems[pb])

        for _i in range(NBUF):
            pltpu.make_async_copy(
                bufs[0], acc.at[dst_v.at[0]], ssem).wait()

    plsc.subcore_barrier()

    @pl.loop(0, ROWS_T // CHUNK)
    def _(k):
        pltpu.sync_copy(acc.at[pl.ds(s * ROWS_T + k * CHUNK, CHUNK)], bufs[0])
        pltpu.sync_copy(
            bufs[0], part.at[c, pl.ds(s * ROWS_T + k * CHUNK, CHUNK)])


def _mm_scale_body(x_ref, w_ref, deg_ref, o_ref):
    dinv = lax.rsqrt(deg_ref[0] + deg_ref[1] + 1.0)
    o_ref[...] = dinv * jnp.dot(
        x_ref[...], w_ref[...], preferred_element_type=jnp.float32)


_mm_scale = pl.pallas_call(
    _mm_scale_body,
    grid=(GRID,),
    in_specs=[
        pl.BlockSpec((RB, D), lambda i: (i, 0)),
        pl.BlockSpec((D, D), lambda i: (0, 0)),
        pl.BlockSpec((NC, RB, D), lambda i: (0, i, 0)),
    ],
    out_specs=pl.BlockSpec((RB, D), lambda i: (i, 0)),
    out_shape=jax.ShapeDtypeStruct((NP, D), jnp.float32),
)


def _combine_mm_body(p_ref, g_ref, deg_ref, b_ref, w_ref, h_ref, g2_ref):
    dinv = lax.rsqrt(deg_ref[0] + deg_ref[1] + 1.0)
    h = jnp.maximum(
        dinv * (p_ref[0] + p_ref[1] + g_ref[...]) + b_ref[...], 0.0)
    h_ref[...] = h
    g2_ref[...] = dinv * jnp.dot(
        h, w_ref[...], preferred_element_type=jnp.float32)


_combine_mm = pl.pallas_call(
    _combine_mm_body,
    grid=(GRID,),
    in_specs=[
        pl.BlockSpec((NC, RB, D), lambda i: (0, i, 0)),
        pl.BlockSpec((RB, D), lambda i: (i, 0)),
        pl.BlockSpec((NC, RB, D), lambda i: (0, i, 0)),
        pl.BlockSpec((1, D), lambda i: (0, 0)),
        pl.BlockSpec((D, D), lambda i: (0, 0)),
    ],
    out_specs=[
        pl.BlockSpec((RB, D), lambda i: (i, 0)),
        pl.BlockSpec((RB, D), lambda i: (i, 0)),
    ],
    out_shape=[
        jax.ShapeDtypeStruct((NP, D), jnp.float32),
        jax.ShapeDtypeStruct((NP, D), jnp.float32),
    ],
)


def _final_body(p_ref, g_ref, deg_ref, b_ref, h1_ref, o_ref):
    dinv = lax.rsqrt(deg_ref[0] + deg_ref[1] + 1.0)
    h2 = jnp.maximum(
        dinv * (p_ref[0] + p_ref[1] + g_ref[...]) + b_ref[...], 0.0)
    o_ref[...] = (h1_ref[...] + h2) * 0.5


_final = pl.pallas_call(
    _final_body,
    grid=(GRID,),
    in_specs=[
        pl.BlockSpec((NC, RB, D), lambda i: (0, i, 0)),
        pl.BlockSpec((RB, D), lambda i: (i, 0)),
        pl.BlockSpec((NC, RB, D), lambda i: (0, i, 0)),
        pl.BlockSpec((1, D), lambda i: (0, 0)),
        pl.BlockSpec((RB, D), lambda i: (i, 0)),
    ],
    out_specs=pl.BlockSpec((RB, D), lambda i: (i, 0)),
    out_shape=jax.ShapeDtypeStruct((NP, D), jnp.float32),
)


def kernel(x, edge_index, edge_weight, W1, b1, W2, b2):
    pad = E_PAD - E
    pad_ids = jnp.arange(pad, dtype=jnp.int32) % N
    srcT = jnp.concatenate([edge_index[0], pad_ids]).reshape(NW, CPT, CHUNK)
    dstT = jnp.concatenate([edge_index[1], pad_ids]).reshape(NW, CPT, CHUNK)
    wT = jnp.pad(edge_weight, (0, pad)).reshape(NW, CPT, CHUNK)
    x_pad = jnp.pad(x, ((0, NP - N), (0, 0)))
    b1r = b1.reshape(1, D)
    b2r = b2.reshape(1, D)

    deg_b = _deg_dinv(dstT, wT)
    g1 = _mm_scale(x_pad, W1, deg_b)
    p1 = _agg(g1, srcT, dstT, wT)
    h1, g2 = _combine_mm(p1, g1, deg_b, b1r, W2)
    p2 = _agg(g2, srcT, dstT, wT)
    out = _final(p2, g2, deg_b, b2r, h1)
    return out[:N]
